# baseline (speedup 1.0000x reference)
import functools

import jax
import jax.numpy as jnp
from jax import lax
from jax.experimental import pallas as pl
from jax.experimental.pallas import tpu as pltpu
from jax.experimental.pallas import tpu_sc as plsc

NC = 2
NS = 16
NW = NC * NS
CHUNK = 128
PAD_ROWS = 16

_MESH = plsc.VectorSubcoreMesh(core_axis_name="c", subcore_axis_name="s")


def _pad_up(e, mult):
  return ((e + mult - 1) // mult) * mult


def _pad_edges(ei, n_src):
  e = ei.shape[1]
  ep = _pad_up(e, NW * CHUNK)
  pad = ep - e
  i = jnp.arange(pad, dtype=jnp.int32)
  srcp = jnp.concatenate([ei[0].astype(jnp.int32), n_src + (i % PAD_ROWS)])
  dstp = jnp.concatenate([ei[1].astype(jnp.int32), i % PAD_ROWS])
  return srcp, dstp, ep


def _zero_vmem_2d(buf, rows, d):
  z = jnp.zeros((16,), jnp.float32)

  def body(r, _):
    for k in range(d // 16):
      buf[r, pl.ds(16 * k, 16)] = z
    return 0

  lax.fori_loop(0, rows, body, 0)


def _zero_vmem_1d(buf, n):
  z = jnp.zeros((16,), jnp.float32)

  def body(r, _):
    buf[pl.ds(r * 16, 16)] = z
    return 0

  lax.fori_loop(0, n // 16, body, 0)


@functools.partial(jax.jit, static_argnames=("n_dst", "d", "e_pad", "e_real"))
def _sc_seg_sum(table, srcp, dstp, *, n_dst, d, e_pad, e_real):
  per_w = e_pad // NW
  n_chunks = per_w // CHUNK
  n_dst_pad = _pad_up(n_dst, NS * CHUNK)
  rows_per_tile = n_dst_pad // NS
  n_copies = rows_per_tile // CHUNK
  deg_pad = n_dst_pad
  deg_per_tile = rows_per_tile

  def body(table_hbm, srcp_hbm, dstp_hbm, acc_hbm, deg_hbm,
           idx_s, idx_d, rows0, rows1, vals, acc_sh, deg_sh,
           sem0, sem1, ssem0, ssem1):
    cid = lax.axis_index("c")
    sid = lax.axis_index("s")
    wid = cid * NS + sid

    pltpu.sync_copy(srcp_hbm.at[wid], idx_s)
    pltpu.sync_copy(dstp_hbm.at[wid], idx_d)

    _zero_vmem_2d(rows0, CHUNK, d)
    _zero_vmem_1d(vals, CHUNK)
    for j in range(n_copies):
      pltpu.sync_copy(rows0,
                      acc_sh.at[pl.ds(sid * rows_per_tile + j * CHUNK, CHUNK)])
    for j in range(deg_per_tile // CHUNK):
      pltpu.sync_copy(vals, deg_sh.at[pl.ds(sid * deg_per_tile + j * CHUNK, CHUNK)])

    bufs = [rows0, rows1]
    sems = [sem0, sem1]
    ssems = [ssem0, ssem1]
    descs = [None, None]
    sdescs = [None, None]
    descs[0] = pltpu.async_copy(table_hbm.at[idx_s.at[0]], rows0, sem0)
    plsc.subcore_barrier()

    iota = lax.iota(jnp.int32, 16)
    for c in range(n_chunks):
      b = c % 2
      descs[b].wait()
      if c + 1 < n_chunks:
        if sdescs[1 - b] is not None:
          sdescs[1 - b].wait()
          sdescs[1 - b] = None
        descs[1 - b] = pltpu.async_copy(table_hbm.at[idx_s.at[c + 1]],
                                        bufs[1 - b], sems[1 - b])
      base = wid * per_w + c * CHUNK
      for j in range(CHUNK // 16):
        m = (base + 16 * j + iota) < e_real
        vals[pl.ds(16 * j, 16)] = jnp.where(m, 1.0, 0.0)
      sdescs[b] = pltpu.async_copy(bufs[b], acc_sh.at[idx_d.at[c]], ssems[b],
                                   add=True)
      pltpu.sync_copy(vals, deg_sh.at[idx_d.at[c]], add=True)
    for sd in sdescs:
      if sd is not None:
        sd.wait()
    plsc.subcore_barrier()

    pltpu.sync_copy(
        acc_sh.at[pl.ds(sid * rows_per_tile, rows_per_tile)],
        acc_hbm.at[cid, pl.ds(sid * rows_per_tile, rows_per_tile)])
    pltpu.sync_copy(
        deg_sh.at[pl.ds(sid * deg_per_tile, deg_per_tile)],
        deg_hbm.at[cid, pl.ds(sid * deg_per_tile, deg_per_tile)])

  fn = pl.kernel(
      body,
      out_type=(jax.ShapeDtypeStruct((NC, n_dst_pad, d), jnp.float32),
                jax.ShapeDtypeStruct((NC, deg_pad), jnp.float32)),
      mesh=_MESH,
      compiler_params=pltpu.CompilerParams(needs_layout_passes=False,
                                           use_tc_tiling_on_sc=(d % 128 == 0)),
      scratch_types=[
          pltpu.VMEM((n_chunks, CHUNK), jnp.int32),
          pltpu.VMEM((n_chunks, CHUNK), jnp.int32),
          pltpu.VMEM((CHUNK, d), jnp.float32),
          pltpu.VMEM((CHUNK, d), jnp.float32),
          pltpu.VMEM((CHUNK,), jnp.float32),
          pltpu.VMEM_SHARED((n_dst_pad, d), jnp.float32),
          pltpu.VMEM_SHARED((deg_pad,), jnp.float32),
          pltpu.SemaphoreType.DMA,
          pltpu.SemaphoreType.DMA,
          pltpu.SemaphoreType.DMA,
          pltpu.SemaphoreType.DMA,
      ],
  )
  return fn(table, srcp.reshape(NW, n_chunks, CHUNK),
            dstp.reshape(NW, n_chunks, CHUNK))


@functools.partial(jax.jit, static_argnames=("n_dst_a", "n_dst_b", "d",
                                             "e_pad_a", "e_real_a",
                                             "e_pad_b", "e_real_b"))
def _sc_seg_sum2(table_a, srcp_a, dstp_a, table_b, srcp_b, dstp_b, *,
                 n_dst_a, n_dst_b, d, e_pad_a, e_real_a, e_pad_b, e_real_b):
  per_w_a = e_pad_a // NW
  per_w_b = e_pad_b // NW
  nc_a = per_w_a // CHUNK
  nc_b = per_w_b // CHUNK
  pad_a = _pad_up(n_dst_a, NS * CHUNK)
  pad_b = _pad_up(n_dst_b, NS * CHUNK)
  rpt_a = pad_a // NS
  rpt_b = pad_b // NS

  def body(ta_hbm, sa_hbm, da_hbm, tb_hbm, sb_hbm, db_hbm,
           acca_hbm, dega_hbm, accb_hbm, degb_hbm,
           idx_sa, idx_da, idx_sb, idx_db, rows0, rows1, vals,
           acca_sh, dega_sh, accb_sh, degb_sh, sem0, sem1, ssem0, ssem1):
    cid = lax.axis_index("c")
    sid = lax.axis_index("s")
    wid = cid * NS + sid

    pltpu.sync_copy(sa_hbm.at[wid], idx_sa)
    pltpu.sync_copy(da_hbm.at[wid], idx_da)
    pltpu.sync_copy(sb_hbm.at[wid], idx_sb)
    pltpu.sync_copy(db_hbm.at[wid], idx_db)

    _zero_vmem_2d(rows0, CHUNK, d)
    _zero_vmem_1d(vals, CHUNK)
    for j in range(rpt_a // CHUNK):
      pltpu.sync_copy(rows0, acca_sh.at[pl.ds(sid * rpt_a + j * CHUNK, CHUNK)])
      pltpu.sync_copy(vals, dega_sh.at[pl.ds(sid * rpt_a + j * CHUNK, CHUNK)])
    for j in range(rpt_b // CHUNK):
      pltpu.sync_copy(rows0, accb_sh.at[pl.ds(sid * rpt_b + j * CHUNK, CHUNK)])
      pltpu.sync_copy(vals, degb_sh.at[pl.ds(sid * rpt_b + j * CHUNK, CHUNK)])

    work = ([(ta_hbm, idx_sa, idx_da, acca_sh, dega_sh, per_w_a, e_real_a, c)
             for c in range(nc_a)] +
            [(tb_hbm, idx_sb, idx_db, accb_sh, degb_sh, per_w_b, e_real_b, c)
             for c in range(nc_b)])
    bufs = [rows0, rows1]
    sems = [sem0, sem1]
    ssems = [ssem0, ssem1]
    descs = [None, None]
    sdescs = [None, None]
    t0, s0, _, _, _, _, _, _ = work[0]
    descs[0] = pltpu.async_copy(t0.at[s0.at[0]], rows0, sem0)
    plsc.subcore_barrier()

    iota = lax.iota(jnp.int32, 16)
    for i, (tbl, isx, idx, acc_sh, deg_sh, per_w, e_real, c) in enumerate(work):
      b = i % 2
      descs[b].wait()
      if i + 1 < len(work):
        if sdescs[1 - b] is not None:
          sdescs[1 - b].wait()
          sdescs[1 - b] = None
        tn, sn, _, _, _, _, _, cn = work[i + 1]
        descs[1 - b] = pltpu.async_copy(tn.at[sn.at[cn]], bufs[1 - b],
                                        sems[1 - b])
      base = wid * per_w + c * CHUNK
      for j in range(CHUNK // 16):
        m = (base + 16 * j + iota) < e_real
        vals[pl.ds(16 * j, 16)] = jnp.where(m, 1.0, 0.0)
      sdescs[b] = pltpu.async_copy(bufs[b], acc_sh.at[idx.at[c]], ssems[b],
                                   add=True)
      pltpu.sync_copy(vals, deg_sh.at[idx.at[c]], add=True)
    for sd in sdescs:
      if sd is not None:
        sd.wait()
    plsc.subcore_barrier()

    pltpu.sync_copy(acca_sh.at[pl.ds(sid * rpt_a, rpt_a)],
                    acca_hbm.at[cid, pl.ds(sid * rpt_a, rpt_a)])
    pltpu.sync_copy(dega_sh.at[pl.ds(sid * rpt_a, rpt_a)],
                    dega_hbm.at[cid, pl.ds(sid * rpt_a, rpt_a)])
    pltpu.sync_copy(accb_sh.at[pl.ds(sid * rpt_b, rpt_b)],
                    accb_hbm.at[cid, pl.ds(sid * rpt_b, rpt_b)])
    pltpu.sync_copy(degb_sh.at[pl.ds(sid * rpt_b, rpt_b)],
                    degb_hbm.at[cid, pl.ds(sid * rpt_b, rpt_b)])

  fn = pl.kernel(
      body,
      out_type=(jax.ShapeDtypeStruct((NC, pad_a, d), jnp.float32),
                jax.ShapeDtypeStruct((NC, pad_a), jnp.float32),
                jax.ShapeDtypeStruct((NC, pad_b, d), jnp.float32),
                jax.ShapeDtypeStruct((NC, pad_b), jnp.float32)),
      mesh=_MESH,
      compiler_params=pltpu.CompilerParams(needs_layout_passes=False,
                                           use_tc_tiling_on_sc=(d % 128 == 0)),
      scratch_types=[
          pltpu.VMEM((nc_a, CHUNK), jnp.int32),
          pltpu.VMEM((nc_a, CHUNK), jnp.int32),
          pltpu.VMEM((nc_b, CHUNK), jnp.int32),
          pltpu.VMEM((nc_b, CHUNK), jnp.int32),
          pltpu.VMEM((CHUNK, d), jnp.float32),
          pltpu.VMEM((CHUNK, d), jnp.float32),
          pltpu.VMEM((CHUNK,), jnp.float32),
          pltpu.VMEM_SHARED((pad_a, d), jnp.float32),
          pltpu.VMEM_SHARED((pad_a,), jnp.float32),
          pltpu.VMEM_SHARED((pad_b, d), jnp.float32),
          pltpu.VMEM_SHARED((pad_b,), jnp.float32),
          pltpu.SemaphoreType.DMA,
          pltpu.SemaphoreType.DMA,
          pltpu.SemaphoreType.DMA,
          pltpu.SemaphoreType.DMA,
      ],
  )
  return fn(table_a, srcp_a.reshape(NW, nc_a, CHUNK),
            dstp_a.reshape(NW, nc_a, CHUNK),
            table_b, srcp_b.reshape(NW, nc_b, CHUNK),
            dstp_b.reshape(NW, nc_b, CHUNK))


@functools.partial(jax.jit, static_argnames=("n_dst", "eg_pad", "eg_real",
                                             "ea_pad", "ea_real"))
def _sc_gat_agg(table, ls_pad, ld, mvec, sg, dg, sa, da, *,
                n_dst, eg_pad, eg_real, ea_pad, ea_real):
  d = 128
  per_w_g = eg_pad // NS
  per_w_a = ea_pad // NS
  nc_g = per_w_g // CHUNK
  nc_a = per_w_a // CHUNK
  nc_max = max(nc_g, nc_a)
  n_dst_pad = _pad_up(n_dst, NS * 8)
  den_out_pad = _pad_up(n_dst, NS * CHUNK)
  rpt = n_dst_pad // NS
  dpt = den_out_pad // NS
  z_full = rpt // CHUNK
  z_rem = rpt % CHUNK

  def body(hs_hbm, ls_hbm, ld_hbm, m_hbm, sg_hbm, dg_hbm,
           sa_hbm, da_hbm, acc_hbm, den_hbm,
           idx_s, idx_d, rows0, rows1, pbuf, ls_v, ld_v, m_v,
           acc_sh, den_sh, sem0, sem1):
    cid = lax.axis_index("c")
    sid = lax.axis_index("s")

    _zero_vmem_2d(rows0, CHUNK, d)
    _zero_vmem_1d(pbuf, CHUNK)
    for j in range(z_full):
      pltpu.sync_copy(rows0, acc_sh.at[pl.ds(sid * rpt + j * CHUNK, CHUNK)])
    if z_rem:
      pltpu.sync_copy(rows0.at[pl.ds(0, z_rem)],
                      acc_sh.at[pl.ds(sid * rpt + z_full * CHUNK, z_rem)])

    @pl.when(cid == 0)
    def _zden():
      for j in range(dpt // CHUNK):
        pltpu.sync_copy(pbuf, den_sh.at[pl.ds(sid * dpt + j * CHUNK, CHUNK)])

    plsc.subcore_barrier()

    bufs = [rows0, rows1]
    sems = [sem0, sem1]
    iota = lax.iota(jnp.int32, 16)

    @pl.when(cid == 0)
    def _gat():
      pltpu.sync_copy(sg_hbm.at[sid], idx_s.at[pl.ds(0, nc_g)])
      pltpu.sync_copy(dg_hbm.at[sid], idx_d.at[pl.ds(0, nc_g)])
      pltpu.sync_copy(ls_hbm, ls_v)
      pltpu.sync_copy(ld_hbm, ld_v)
      pltpu.sync_copy(m_hbm, m_v)
      m16 = m_v[...]
      descs = [None, None]
      descs[0] = pltpu.async_copy(hs_hbm.at[idx_s.at[0]], rows0, sem0)
      for c in range(nc_g):
        b = c % 2
        descs[b].wait()
        if c + 1 < nc_g:
          descs[1 - b] = pltpu.async_copy(hs_hbm.at[idx_s.at[c + 1]],
                                          bufs[1 - b], sems[1 - b])
        base = sid * per_w_g + c * CHUNK
        for j in range(CHUNK // 16):
          sv = idx_s[c, pl.ds(16 * j, 16)]
          dv = idx_d[c, pl.ds(16 * j, 16)]
          t = plsc.load_gather(ls_v, [sv]) + plsc.load_gather(ld_v, [dv])
          a = jnp.maximum(t, 0.2 * t)
          pv = jnp.exp(a - m16)
          msk = (base + 16 * j + iota) < eg_real
          pbuf[pl.ds(16 * j, 16)] = jnp.where(msk, pv, 0.0)
        rows_c = bufs[b]

        def scale_row(r, _):
          pb = plsc.load_gather(pbuf, [jnp.full((16,), r, jnp.int32)])
          for k in range(d // 16):
            rows_c[r, pl.ds(16 * k, 16)] = rows_c[r, pl.ds(16 * k, 16)] * pb
          return 0

        lax.fori_loop(0, CHUNK, scale_row, 0)
        pltpu.sync_copy(rows_c, acc_sh.at[idx_d.at[c]], add=True)
        pltpu.sync_copy(pbuf, den_sh.at[idx_d.at[c]], add=True)

    @pl.when(cid == 1)
    def _agg():
      pltpu.sync_copy(sa_hbm.at[sid], idx_s.at[pl.ds(0, nc_a)])
      pltpu.sync_copy(da_hbm.at[sid], idx_d.at[pl.ds(0, nc_a)])
      descs = [None, None]
      descs[0] = pltpu.async_copy(hs_hbm.at[idx_s.at[0]], rows0, sem0)
      for c in range(nc_a):
        b = c % 2
        descs[b].wait()
        if c + 1 < nc_a:
          descs[1 - b] = pltpu.async_copy(hs_hbm.at[idx_s.at[c + 1]],
                                          bufs[1 - b], sems[1 - b])
        base = sid * per_w_a + c * CHUNK
        pltpu.sync_copy(bufs[b], acc_sh.at[idx_d.at[c]], add=True)

    plsc.subcore_barrier()
    pltpu.sync_copy(acc_sh.at[pl.ds(sid * rpt, rpt)],
                    acc_hbm.at[cid, pl.ds(sid * rpt, rpt)])

    @pl.when(cid == 0)
    def _oden():
      pltpu.sync_copy(den_sh.at[pl.ds(sid * dpt, dpt)],
                      den_hbm.at[0, pl.ds(sid * dpt, dpt)])

  fn = pl.kernel(
      body,
      out_type=(jax.ShapeDtypeStruct((NC, n_dst_pad, d), jnp.float32),
                jax.ShapeDtypeStruct((1, den_out_pad), jnp.float32)),
      mesh=_MESH,
      compiler_params=pltpu.CompilerParams(needs_layout_passes=False),
      scratch_types=[
          pltpu.VMEM((nc_max, CHUNK), jnp.int32),
          pltpu.VMEM((nc_max, CHUNK), jnp.int32),
          pltpu.VMEM((CHUNK, d), jnp.float32),
          pltpu.VMEM((CHUNK, d), jnp.float32),
          pltpu.VMEM((CHUNK,), jnp.float32),
          pltpu.VMEM((ls_pad.shape[0],), jnp.float32),
          pltpu.VMEM((ld.shape[0],), jnp.float32),
          pltpu.VMEM((16,), jnp.float32),
          pltpu.VMEM_SHARED((n_dst_pad, d), jnp.float32),
          pltpu.VMEM_SHARED((den_out_pad,), jnp.float32),
          pltpu.SemaphoreType.DMA,
          pltpu.SemaphoreType.DMA,
      ],
  )
  return fn(table, ls_pad, ld, mvec,
            sg.reshape(NS, nc_g, CHUNK), dg.reshape(NS, nc_g, CHUNK),
            sa.reshape(NS, nc_a, CHUNK), da.reshape(NS, nc_a, CHUNK))


@functools.partial(jax.jit, static_argnames=("n_dst", "e_pad", "e_real"))
def _sc_gat(hs_pad, ls_pad, ld, mvec, srcp, dstp, *, n_dst, e_pad, e_real):
  d = 128
  n_src_pad = hs_pad.shape[0]
  per_w = e_pad // NW
  n_chunks = per_w // CHUNK
  n_dst_pad = _pad_up(n_dst, NS * CHUNK)
  rows_per_tile = n_dst_pad // NS
  n_copies = rows_per_tile // CHUNK
  den_pad = n_dst_pad
  den_per_tile = rows_per_tile

  def body(hs_hbm, ls_hbm, ld_hbm, m_hbm, srcp_hbm, dstp_hbm,
           num_hbm, den_hbm,
           idx_s, idx_d, rows0, rows1, pbuf, ls_v, ld_v, m_v,
           num_sh, den_sh, sem0, sem1, ssem0, ssem1):
    cid = lax.axis_index("c")
    sid = lax.axis_index("s")
    wid = cid * NS + sid

    pltpu.sync_copy(srcp_hbm.at[wid], idx_s)
    pltpu.sync_copy(dstp_hbm.at[wid], idx_d)
    pltpu.sync_copy(ls_hbm, ls_v)
    pltpu.sync_copy(ld_hbm, ld_v)
    pltpu.sync_copy(m_hbm, m_v)

    _zero_vmem_2d(rows0, CHUNK, d)
    _zero_vmem_1d(pbuf, CHUNK)
    for j in range(n_copies):
      pltpu.sync_copy(rows0,
                      num_sh.at[pl.ds(sid * rows_per_tile + j * CHUNK, CHUNK)])
    for j in range(den_per_tile // CHUNK):
      pltpu.sync_copy(pbuf, den_sh.at[pl.ds(sid * den_per_tile + j * CHUNK, CHUNK)])

    bufs = [rows0, rows1]
    sems = [sem0, sem1]
    ssems = [ssem0, ssem1]
    descs = [None, None]
    sdescs = [None, None]
    descs[0] = pltpu.async_copy(hs_hbm.at[idx_s.at[0]], rows0, sem0)
    plsc.subcore_barrier()

    iota = lax.iota(jnp.int32, 16)
    m16 = m_v[...]
    for c in range(n_chunks):
      b = c % 2
      descs[b].wait()
      if c + 1 < n_chunks:
        if sdescs[1 - b] is not None:
          sdescs[1 - b].wait()
          sdescs[1 - b] = None
        descs[1 - b] = pltpu.async_copy(hs_hbm.at[idx_s.at[c + 1]],
                                        bufs[1 - b], sems[1 - b])
      base = wid * per_w + c * CHUNK
      for j in range(CHUNK // 16):
        sv = idx_s[c, pl.ds(16 * j, 16)]
        dv = idx_d[c, pl.ds(16 * j, 16)]
        lsg = plsc.load_gather(ls_v, [sv])
        ldg = plsc.load_gather(ld_v, [dv])
        t = lsg + ldg
        a = jnp.maximum(t, 0.2 * t)
        pv = jnp.exp(a - m16)
        m = (base + 16 * j + iota) < e_real
        pbuf[pl.ds(16 * j, 16)] = jnp.where(m, pv, 0.0)

      rows_c = bufs[b]

      def scale_row(g, _):
        for u in range(2):
          r = 2 * g + u
          pb = plsc.load_gather(pbuf, [jnp.full((16,), r, jnp.int32)])
          for k in range(d // 16):
            rows_c[r, pl.ds(16 * k, 16)] = rows_c[r, pl.ds(16 * k, 16)] * pb
        return 0

      lax.fori_loop(0, CHUNK // 2, scale_row, 0)
      sdescs[b] = pltpu.async_copy(rows_c, num_sh.at[idx_d.at[c]], ssems[b],
                                   add=True)
      pltpu.sync_copy(pbuf, den_sh.at[idx_d.at[c]], add=True)
    for sd in sdescs:
      if sd is not None:
        sd.wait()
    plsc.subcore_barrier()

    pltpu.sync_copy(
        num_sh.at[pl.ds(sid * rows_per_tile, rows_per_tile)],
        num_hbm.at[cid, pl.ds(sid * rows_per_tile, rows_per_tile)])
    pltpu.sync_copy(
        den_sh.at[pl.ds(sid * den_per_tile, den_per_tile)],
        den_hbm.at[cid, pl.ds(sid * den_per_tile, den_per_tile)])

  fn = pl.kernel(
      body,
      out_type=(jax.ShapeDtypeStruct((NC, n_dst_pad, d), jnp.float32),
                jax.ShapeDtypeStruct((NC, den_pad), jnp.float32)),
      mesh=_MESH,
      compiler_params=pltpu.CompilerParams(needs_layout_passes=False),
      scratch_types=[
          pltpu.VMEM((n_chunks, CHUNK), jnp.int32),
          pltpu.VMEM((n_chunks, CHUNK), jnp.int32),
          pltpu.VMEM((CHUNK, d), jnp.float32),
          pltpu.VMEM((CHUNK, d), jnp.float32),
          pltpu.VMEM((CHUNK,), jnp.float32),
          pltpu.VMEM((ls_pad.shape[0],), jnp.float32),
          pltpu.VMEM((ld.shape[0],), jnp.float32),
          pltpu.VMEM((16,), jnp.float32),
          pltpu.VMEM_SHARED((n_dst_pad, d), jnp.float32),
          pltpu.VMEM_SHARED((den_pad,), jnp.float32),
          pltpu.SemaphoreType.DMA,
          pltpu.SemaphoreType.DMA,
          pltpu.SemaphoreType.DMA,
          pltpu.SemaphoreType.DMA,
      ],
  )
  return fn(hs_pad, ls_pad, ld, mvec, srcp.reshape(NW, n_chunks, CHUNK),
            dstp.reshape(NW, n_chunks, CHUNK))


@functools.partial(jax.jit, static_argnames=("n_dst", "e_pad", "e_real"))
def _sc_leaf_pool(a4a, a4b, ytab, g1a, g1b, b1, srcp, dstp, *,
                  n_dst, e_pad, e_real):
  d = 32
  per_w = e_pad // NW
  n_chunks = per_w // CHUNK
  n_dst_pad = _pad_up(n_dst, NS * CHUNK)
  rpt = n_dst_pad // NS

  def body(a4a_hbm, a4b_hbm, y_hbm, g1a_hbm, g1b_hbm, b1_hbm,
           srcp_hbm, dstp_hbm, acc_hbm, deg_hbm,
           idx_s, idx_d, ra0, ra1, rb0, rb1, ry0, ry1, wbuf, mbuf,
           g1a_v, g1b_v, b1_v, acc_sh, deg_sh,
           sa0, sa1, sb0, sb1, sy0, sy1):
    cid = lax.axis_index("c")
    sid = lax.axis_index("s")
    wid = cid * NS + sid

    pltpu.sync_copy(srcp_hbm.at[wid], idx_s)
    pltpu.sync_copy(dstp_hbm.at[wid], idx_d)
    pltpu.sync_copy(g1a_hbm, g1a_v)
    pltpu.sync_copy(g1b_hbm, g1b_v)
    pltpu.sync_copy(b1_hbm, b1_v)

    _zero_vmem_2d(ra0, CHUNK, d)
    _zero_vmem_1d(mbuf, CHUNK)
    for j in range(rpt // CHUNK):
      pltpu.sync_copy(ra0, acc_sh.at[pl.ds(sid * rpt + j * CHUNK, CHUNK)])
      pltpu.sync_copy(mbuf, deg_sh.at[pl.ds(sid * rpt + j * CHUNK, CHUNK)])

    abufs = [ra0, ra1]
    bbufs = [rb0, rb1]
    ybufs = [ry0, ry1]
    asems = [sa0, sa1]
    bsems = [sb0, sb1]
    ysems = [sy0, sy1]
    descs = [None, None]

    def start(c, b):
      descs[b] = (pltpu.async_copy(a4a_hbm.at[idx_s.at[c]], abufs[b], asems[b]),
                  pltpu.async_copy(a4b_hbm.at[idx_s.at[c]], bbufs[b], bsems[b]),
                  pltpu.async_copy(y_hbm.at[idx_s.at[c]], ybufs[b], ysems[b]))

    start(0, 0)
    plsc.subcore_barrier()

    iota = lax.iota(jnp.int32, 16)
    one = jnp.ones((16,), jnp.float32)
    b1lo = b1_v[pl.ds(0, 16)]
    b1hi = b1_v[pl.ds(16, 16)]
    for c in range(n_chunks):
      b = c % 2
      for dd in descs[b]:
        dd.wait()
      if c + 1 < n_chunks:
        start(c + 1, 1 - b)
      base = wid * per_w + c * CHUNK
      for j in range(CHUNK // 16):
        sv = idx_s[c, pl.ds(16 * j, 16)]
        dg = plsc.load_gather(g1a_v, [sv]) + plsc.load_gather(g1b_v, [sv])
        inv = 1.0 / jnp.maximum(dg, 1.0)
        msk = (base + 16 * j + iota) < e_real
        m = jnp.where(msk, 1.0, 0.0)
        mbuf[pl.ds(16 * j, 16)] = m
        wbuf[pl.ds(16 * j, 16)] = inv * m

      ra, rb, ry = abufs[b], bbufs[b], ybufs[b]

      def combine_row(r, _):
        ridx = jnp.full((16,), r, jnp.int32)
        w = plsc.load_gather(wbuf, [ridx])
        m = plsc.load_gather(mbuf, [ridx])
        lo = pl.ds(0, 16)
        hi = pl.ds(16, 16)
        ra[r, lo] = ((ra[r, lo] + rb[r, lo]) * w
                     + (ry[r, lo] + b1lo) * m)
        ra[r, hi] = ((ra[r, hi] + rb[r, hi]) * w
                     + (ry[r, hi] + b1hi) * m)
        return 0

      lax.fori_loop(0, CHUNK, combine_row, 0)
      pltpu.sync_copy(ra, acc_sh.at[idx_d.at[c]], add=True)
      pltpu.sync_copy(mbuf, deg_sh.at[idx_d.at[c]], add=True)
    plsc.subcore_barrier()

    pltpu.sync_copy(acc_sh.at[pl.ds(sid * rpt, rpt)],
                    acc_hbm.at[cid, pl.ds(sid * rpt, rpt)])
    pltpu.sync_copy(deg_sh.at[pl.ds(sid * rpt, rpt)],
                    deg_hbm.at[cid, pl.ds(sid * rpt, rpt)])

  fn = pl.kernel(
      body,
      out_type=(jax.ShapeDtypeStruct((NC, n_dst_pad, d), jnp.float32),
                jax.ShapeDtypeStruct((NC, n_dst_pad), jnp.float32)),
      mesh=_MESH,
      compiler_params=pltpu.CompilerParams(needs_layout_passes=False,
                                           use_tc_tiling_on_sc=False),
      scratch_types=[
          pltpu.VMEM((n_chunks, CHUNK), jnp.int32),
          pltpu.VMEM((n_chunks, CHUNK), jnp.int32),
          pltpu.VMEM((CHUNK, d), jnp.float32),
          pltpu.VMEM((CHUNK, d), jnp.float32),
          pltpu.VMEM((CHUNK, d), jnp.float32),
          pltpu.VMEM((CHUNK, d), jnp.float32),
          pltpu.VMEM((CHUNK, d), jnp.float32),
          pltpu.VMEM((CHUNK, d), jnp.float32),
          pltpu.VMEM((CHUNK,), jnp.float32),
          pltpu.VMEM((CHUNK,), jnp.float32),
          pltpu.VMEM((g1a.shape[0],), jnp.float32),
          pltpu.VMEM((g1b.shape[0],), jnp.float32),
          pltpu.VMEM((32,), jnp.float32),
          pltpu.VMEM_SHARED((n_dst_pad, d), jnp.float32),
          pltpu.VMEM_SHARED((n_dst_pad,), jnp.float32),
          pltpu.SemaphoreType.DMA,
          pltpu.SemaphoreType.DMA,
          pltpu.SemaphoreType.DMA,
          pltpu.SemaphoreType.DMA,
          pltpu.SemaphoreType.DMA,
          pltpu.SemaphoreType.DMA,
      ],
  )
  return fn(a4a, a4b, ytab, g1a, g1b, b1,
            srcp.reshape(NW, n_chunks, CHUNK), dstp.reshape(NW, n_chunks, CHUNK))


def _tc_a_body(xrv_ref, xr_ref, ws_ref, wd_ref, as_ref, ad_ref,
               hs_ref, ls_ref, ld_ref, m_ref):
  hs = jnp.dot(xrv_ref[...], ws_ref[...], preferred_element_type=jnp.float32)
  hs_ref[...] = hs
  ls = jnp.sum(hs * as_ref[...], axis=1, keepdims=True)
  ls_ref[...] = ls
  wv = jnp.dot(wd_ref[...], ad_ref[...].T, preferred_element_type=jnp.float32)
  ld = jnp.dot(xr_ref[...], wv, preferred_element_type=jnp.float32)
  ld_ref[...] = ld
  t = jnp.max(ls) + jnp.max(ld)
  m_ref[...] = jnp.full((1, 128), jnp.maximum(t, 0.2 * t), jnp.float32)


@jax.jit
def _tc_a(xrv, xr, ws, wd, att_s, att_d):
  n_rv, dd = xrv.shape
  n_r = xr.shape[0]
  return pl.pallas_call(
      _tc_a_body,
      out_shape=(jax.ShapeDtypeStruct((n_rv, dd), jnp.float32),
                 jax.ShapeDtypeStruct((n_rv, 1), jnp.float32),
                 jax.ShapeDtypeStruct((n_r, 1), jnp.float32),
                 jax.ShapeDtypeStruct((1, 128), jnp.float32)),
  )(xrv, xr, ws, wd, att_s.reshape(1, -1), att_d.reshape(1, -1))


def _tc_b1a_body(na_ref, nb_ref, da_ref, db_ref, bg_ref, rg_ref):
  den = da_ref[...] + db_ref[...]
  rg_ref[...] = (na_ref[...] + nb_ref[...]) / (den + 1e-16) + bg_ref[...]


@jax.jit
def _tc_b1a(num_a, num_b, den_a, den_b, bg):
  n, d = num_a.shape
  blk = 2000
  row = lambda i: (i, 0)
  full = lambda i: (0, 0)
  rspec = pl.BlockSpec((blk, d), row)
  cspec = pl.BlockSpec((blk, 1), row)
  return pl.pallas_call(
      _tc_b1a_body,
      grid=(n // blk,),
      in_specs=[rspec, rspec, cspec, cspec, pl.BlockSpec((1, d), full)],
      out_specs=rspec,
      out_shape=jax.ShapeDtypeStruct((n, d), jnp.float32),
  )(num_a, num_b, den_a, den_b, bg)


def _tc_b1b_body(rg_ref, a1a_ref, a1b_ref, g1a_ref, g1b_ref, wl_ref, wr_ref,
                 b0_ref, r0_ref):
  deg = jnp.maximum(g1a_ref[...] + g1b_ref[...], 1.0)
  m1 = (a1a_ref[...] + a1b_ref[...]) / deg
  r0 = (jnp.dot(m1, wl_ref[...], preferred_element_type=jnp.float32)
        + jnp.dot(rg_ref[...], wr_ref[...], preferred_element_type=jnp.float32)
        + b0_ref[...])
  r0_ref[...] = jnp.maximum(r0, 0.0)


@jax.jit
def _tc_b1b(rg, a1a, a1b, g1a, g1b, wl, wr, b0):
  n, d = rg.shape
  blk = 2000
  row = lambda i: (i, 0)
  full = lambda i: (0, 0)
  rspec = pl.BlockSpec((blk, d), row)
  cspec = pl.BlockSpec((blk, 1), row)
  wspec = pl.BlockSpec((d, d), full)
  return pl.pallas_call(
      _tc_b1b_body,
      grid=(n // blk,),
      in_specs=[rspec, rspec, rspec, cspec, cspec, wspec, wspec,
                pl.BlockSpec((1, d), full)],
      out_specs=rspec,
      out_shape=jax.ShapeDtypeStruct((n, d), jnp.float32),
  )(rg, a1a, a1b, g1a, g1b, wl, wr, b0)


def _tc_b2_body(a2a_ref, a2b_ref, g2a_ref, g2b_ref, a3a_ref, a3b_ref,
                g3a_ref, g3b_ref, xrr_ref,
                rg_ref, a1a_ref, a1b_ref, g1a_ref, g1b_ref,
                wl0_ref, wr0_ref, b0_ref,
                wl_a_ref, wr_a_ref, ba_ref, wl_b_ref, wr_b_ref, bb_ref,
                wl1_ref, wr1_ref, z_ref, y_ref):
  deg2 = jnp.maximum(g2a_ref[...] + g2b_ref[...], 1.0)
  m2 = (a2a_ref[...] + a2b_ref[...]) / deg2
  deg3 = jnp.maximum(g3a_ref[...] + g3b_ref[...], 1.0)
  m3 = (a3a_ref[...] + a3b_ref[...]) / deg3
  xrr = xrr_ref[...]
  rr = (jnp.dot(m2, wl_a_ref[...], preferred_element_type=jnp.float32)
        + jnp.dot(xrr, wr_a_ref[...], preferred_element_type=jnp.float32)
        + ba_ref[...]
        + jnp.dot(m3, wl_b_ref[...], preferred_element_type=jnp.float32)
        + jnp.dot(xrr, wr_b_ref[...], preferred_element_type=jnp.float32)
        + bb_ref[...])
  rr0 = jnp.maximum(0.5 * rr, 0.0)
  z_ref[...] = jnp.dot(rr0, wl1_ref[...], preferred_element_type=jnp.float32)
  deg1 = jnp.maximum(g1a_ref[...] + g1b_ref[...], 1.0)
  m1 = (a1a_ref[...] + a1b_ref[...]) / deg1
  r0 = jnp.maximum(
      jnp.dot(m1, wl0_ref[...], preferred_element_type=jnp.float32)
      + jnp.dot(rg_ref[...], wr0_ref[...], preferred_element_type=jnp.float32)
      + b0_ref[...], 0.0)
  y_ref[...] = jnp.dot(r0, wr1_ref[...], preferred_element_type=jnp.float32)


@jax.jit
def _tc_b2(a2a, a2b, g2a, g2b, a3a, a3b, g3a, g3b, xrr,
           rg, a1a, a1b, g1a, g1b, wl0, wr0, b0,
           wl_a, wr_a, ba, wl_b, wr_b, bb, wl1, wr1):
  n_rr, d = xrr.shape
  n_r = rg.shape[0]
  do = wl1.shape[1]
  return pl.pallas_call(
      _tc_b2_body,
      out_shape=(jax.ShapeDtypeStruct((n_rr, do), jnp.float32),
                 jax.ShapeDtypeStruct((n_r, do), jnp.float32)),
  )(a2a, a2b, g2a, g2b, a3a, a3b, g3a, g3b, xrr,
    rg, a1a, a1b, g1a, g1b, wl0, wr0, b0,
    wl_a, wr_a, ba, wl_b, wr_b, bb, wl1, wr1)


def _tc_b3_body(a4a_ref, a4b_ref, g1a_ref, g1b_ref, y_ref, b1_ref, r1_ref):
  deg = jnp.maximum(g1a_ref[...] + g1b_ref[...], 1.0)
  r1_ref[...] = (a4a_ref[...] + a4b_ref[...]) / deg + y_ref[...] + b1_ref[...]


@jax.jit
def _tc_b3(a4a, a4b, g1a, g1b, y, b1):
  n, do = y.shape
  blk = 2000
  grid = n // blk
  row = lambda i: (i, 0)
  full = lambda i: (0, 0)
  rspec = pl.BlockSpec((blk, do), row)
  cspec = pl.BlockSpec((blk, 1), row)
  return pl.pallas_call(
      _tc_b3_body,
      grid=(grid,),
      in_specs=[rspec, rspec, cspec, cspec, rspec,
                pl.BlockSpec((1, do), full)],
      out_specs=rspec,
      out_shape=jax.ShapeDtypeStruct((n, do), jnp.float32),
  )(a4a, a4b, g1a, g1b, y, b1)


def _tc_b4_body(a5a_ref, a5b_ref, g5a_ref, g5b_ref, o_ref):
  deg = jnp.maximum(g5a_ref[...] + g5b_ref[...], 1.0)
  o_ref[...] = (a5a_ref[...] + a5b_ref[...]) / deg


@jax.jit
def _tc_b4(a5a, a5b, g5a, g5b):
  n, do = a5a.shape
  return pl.pallas_call(
      _tc_b4_body,
      out_shape=jax.ShapeDtypeStruct((n, do), jnp.float32),
  )(a5a, a5b, g5a, g5b)


def _zpad(x, rows=PAD_ROWS):
  return jnp.concatenate([x, jnp.zeros((rows,) + x.shape[1:], x.dtype)], axis=0)


def kernel(x_object, x_room, x_object_virtual, x_room_virtual, x_or, x_rr,
           params, e_ov_o, e_rv_r, e_o_or, e_or_o, e_r_rr, e_rr_r, e_or_rr,
           e_rr_or, e_r_rv):
  pre = params['pre_rv_r']
  s0 = params['sage0']
  s1 = params['sage1']
  n_room = x_room.shape[0]
  n_rr = x_rr.shape[0]
  n_rv = x_room_virtual.shape[0]
  n_or = x_or.shape[0]

  hs, ls, ld, m = _tc_a(x_room_virtual, x_room, pre['Ws'], pre['Wd'],
                        pre['att_s'], pre['att_d'])
  hs_pad = _zpad(hs)
  ls_pad = jnp.concatenate([ls[:, 0], jnp.zeros((PAD_ROWS,), jnp.float32)])
  mvec = m[0, :16]

  src_gat, dst_gat, ep_gat = _pad_edges(e_rv_r, n_rv)
  src_rrr, dst_rrr, ep_rrr = _pad_edges(e_rr_r, n_rr)
  src_r2r, dst_r2r, ep_r2r = _pad_edges(e_r_rr, n_room)
  src_orr, dst_orr, ep_orr = _pad_edges(e_or_rr, n_or)
  src_rv, dst_rv, ep_rv = _pad_edges(e_r_rv, n_room)

  agg1, deg1 = _sc_seg_sum(_zpad(x_rr), src_rrr, dst_rrr,
                           n_dst=n_room, d=128, e_pad=ep_rrr,
                           e_real=e_rr_r.shape[1])
  num, den = _sc_gat(hs_pad, ls_pad, ld[:, 0], mvec, src_gat, dst_gat,
                     n_dst=n_room, e_pad=ep_gat, e_real=e_rv_r.shape[1])

  rg = _tc_b1a(num[0, :n_room], num[1, :n_room],
               den[0, :n_room, None], den[1, :n_room, None],
               pre['b'].reshape(1, -1))

  agg2, deg2, agg3, deg3 = _sc_seg_sum2(
      _zpad(rg), src_r2r, dst_r2r, _zpad(x_or), src_orr, dst_orr,
      n_dst_a=n_rr, n_dst_b=n_rr, d=128,
      e_pad_a=ep_r2r, e_real_a=e_r_rr.shape[1],
      e_pad_b=ep_orr, e_real_b=e_or_rr.shape[1])

  z, y = _tc_b2(agg2[0, :n_rr], agg2[1, :n_rr],
                deg2[0, :n_rr, None], deg2[1, :n_rr, None],
                agg3[0, :n_rr], agg3[1, :n_rr],
                deg3[0, :n_rr, None], deg3[1, :n_rr, None],
                x_rr,
                rg, agg1[0, :n_room], agg1[1, :n_room],
                deg1[0, :n_room, None], deg1[1, :n_room, None],
                s0['rr_r']['Wl'], s0['rr_r']['Wr'],
                s0['rr_r']['b'].reshape(1, -1),
                s0['r_rr']['Wl'], s0['r_rr']['Wr'], s0['r_rr']['b'].reshape(1, -1),
                s0['or_rr']['Wl'], s0['or_rr']['Wr'], s0['or_rr']['b'].reshape(1, -1),
                s1['rr_r']['Wl'], s1['rr_r']['Wr'])

  agg4, _ = _sc_seg_sum(_zpad(z), src_rrr, dst_rrr,
                        n_dst=n_room, d=32, e_pad=ep_rrr,
                        e_real=e_rr_r.shape[1])

  r1 = _tc_b3(agg4[0, :n_room], agg4[1, :n_room],
              deg1[0, :n_room, None], deg1[1, :n_room, None],
              y, s1['rr_r']['b'].reshape(1, -1))

  agg5, deg5 = _sc_seg_sum(_zpad(r1), src_rv, dst_rv,
                           n_dst=n_rv, d=32, e_pad=ep_rv,
                           e_real=e_r_rv.shape[1])

  return _tc_b4(agg5[0, :n_rv], agg5[1, :n_rv],
                deg5[0, :n_rv, None], deg5[1, :n_rv, None])


# speedup vs baseline: 1.0064x; 1.0064x over previous
import functools

import jax
import jax.numpy as jnp
from jax import lax
from jax.experimental import pallas as pl
from jax.experimental.pallas import tpu as pltpu
from jax.experimental.pallas import tpu_sc as plsc

NC = 2
NS = 16
NW = NC * NS
CHUNK = 128
PAD_ROWS = 16

_MESH = plsc.VectorSubcoreMesh(core_axis_name="c", subcore_axis_name="s")


def _pad_up(e, mult):
  return ((e + mult - 1) // mult) * mult


def _pad_edges(ei, n_src):
  e = ei.shape[1]
  ep = _pad_up(e, NW * CHUNK)
  pad = ep - e
  i = jnp.arange(pad, dtype=jnp.int32)
  srcp = jnp.concatenate([ei[0].astype(jnp.int32), n_src + (i % PAD_ROWS)])
  dstp = jnp.concatenate([ei[1].astype(jnp.int32), i % PAD_ROWS])
  return srcp, dstp, ep


def _zero_vmem_2d(buf, rows, d):
  z = jnp.zeros((16,), jnp.float32)

  def body(r, _):
    for k in range(d // 16):
      buf[r, pl.ds(16 * k, 16)] = z
    return 0

  lax.fori_loop(0, rows, body, 0)


def _zero_vmem_1d(buf, n):
  z = jnp.zeros((16,), jnp.float32)

  def body(r, _):
    buf[pl.ds(r * 16, 16)] = z
    return 0

  lax.fori_loop(0, n // 16, body, 0)


@functools.partial(jax.jit, static_argnames=("n_dst", "d", "e_pad", "e_real"))
def _sc_seg_sum(table, srcp, dstp, *, n_dst, d, e_pad, e_real):
  per_w = e_pad // NW
  n_chunks = per_w // CHUNK
  n_dst_pad = _pad_up(n_dst, NS * CHUNK)
  rows_per_tile = n_dst_pad // NS
  n_copies = rows_per_tile // CHUNK
  deg_pad = n_dst_pad
  deg_per_tile = rows_per_tile

  def body(table_hbm, srcp_hbm, dstp_hbm, acc_hbm, deg_hbm,
           idx_s, idx_d, rows0, rows1, vals, acc_sh, deg_sh,
           sem0, sem1, ssem0, ssem1):
    cid = lax.axis_index("c")
    sid = lax.axis_index("s")
    wid = cid * NS + sid

    pltpu.sync_copy(srcp_hbm.at[wid], idx_s)
    pltpu.sync_copy(dstp_hbm.at[wid], idx_d)

    _zero_vmem_2d(rows0, CHUNK, d)
    _zero_vmem_1d(vals, CHUNK)
    for j in range(n_copies):
      pltpu.sync_copy(rows0,
                      acc_sh.at[pl.ds(sid * rows_per_tile + j * CHUNK, CHUNK)])
    for j in range(deg_per_tile // CHUNK):
      pltpu.sync_copy(vals, deg_sh.at[pl.ds(sid * deg_per_tile + j * CHUNK, CHUNK)])

    bufs = [rows0, rows1]
    sems = [sem0, sem1]
    ssems = [ssem0, ssem1]
    descs = [None, None]
    sdescs = [None, None]
    descs[0] = pltpu.async_copy(table_hbm.at[idx_s.at[0]], rows0, sem0)
    plsc.subcore_barrier()

    iota = lax.iota(jnp.int32, 16)
    for c in range(n_chunks):
      b = c % 2
      descs[b].wait()
      if c + 1 < n_chunks:
        if sdescs[1 - b] is not None:
          sdescs[1 - b].wait()
          sdescs[1 - b] = None
        descs[1 - b] = pltpu.async_copy(table_hbm.at[idx_s.at[c + 1]],
                                        bufs[1 - b], sems[1 - b])
      base = wid * per_w + c * CHUNK
      for j in range(CHUNK // 16):
        m = (base + 16 * j + iota) < e_real
        vals[pl.ds(16 * j, 16)] = jnp.where(m, 1.0, 0.0)
      sdescs[b] = pltpu.async_copy(bufs[b], acc_sh.at[idx_d.at[c]], ssems[b],
                                   add=True)
      pltpu.sync_copy(vals, deg_sh.at[idx_d.at[c]], add=True)
    for sd in sdescs:
      if sd is not None:
        sd.wait()
    plsc.subcore_barrier()

    pltpu.sync_copy(
        acc_sh.at[pl.ds(sid * rows_per_tile, rows_per_tile)],
        acc_hbm.at[cid, pl.ds(sid * rows_per_tile, rows_per_tile)])
    pltpu.sync_copy(
        deg_sh.at[pl.ds(sid * deg_per_tile, deg_per_tile)],
        deg_hbm.at[cid, pl.ds(sid * deg_per_tile, deg_per_tile)])

  fn = pl.kernel(
      body,
      out_type=(jax.ShapeDtypeStruct((NC, n_dst_pad, d), jnp.float32),
                jax.ShapeDtypeStruct((NC, deg_pad), jnp.float32)),
      mesh=_MESH,
      compiler_params=pltpu.CompilerParams(needs_layout_passes=False,
                                           use_tc_tiling_on_sc=(d % 128 == 0)),
      scratch_types=[
          pltpu.VMEM((n_chunks, CHUNK), jnp.int32),
          pltpu.VMEM((n_chunks, CHUNK), jnp.int32),
          pltpu.VMEM((CHUNK, d), jnp.float32),
          pltpu.VMEM((CHUNK, d), jnp.float32),
          pltpu.VMEM((CHUNK,), jnp.float32),
          pltpu.VMEM_SHARED((n_dst_pad, d), jnp.float32),
          pltpu.VMEM_SHARED((deg_pad,), jnp.float32),
          pltpu.SemaphoreType.DMA,
          pltpu.SemaphoreType.DMA,
          pltpu.SemaphoreType.DMA,
          pltpu.SemaphoreType.DMA,
      ],
  )
  return fn(table, srcp.reshape(NW, n_chunks, CHUNK),
            dstp.reshape(NW, n_chunks, CHUNK))


@functools.partial(jax.jit, static_argnames=("n_dst_a", "n_dst_b", "d",
                                             "e_pad_a", "e_real_a",
                                             "e_pad_b", "e_real_b"))
def _sc_seg_sum2(table_a, srcp_a, dstp_a, table_b, srcp_b, dstp_b, *,
                 n_dst_a, n_dst_b, d, e_pad_a, e_real_a, e_pad_b, e_real_b):
  per_w_a = e_pad_a // NW
  per_w_b = e_pad_b // NW
  nc_a = per_w_a // CHUNK
  nc_b = per_w_b // CHUNK
  pad_a = _pad_up(n_dst_a, NS * CHUNK)
  pad_b = _pad_up(n_dst_b, NS * CHUNK)
  rpt_a = pad_a // NS
  rpt_b = pad_b // NS

  def body(ta_hbm, sa_hbm, da_hbm, tb_hbm, sb_hbm, db_hbm,
           acca_hbm, dega_hbm, accb_hbm, degb_hbm,
           idx_sa, idx_da, idx_sb, idx_db, rows0, rows1, vals,
           acca_sh, dega_sh, accb_sh, degb_sh, sem0, sem1, ssem0, ssem1):
    cid = lax.axis_index("c")
    sid = lax.axis_index("s")
    wid = cid * NS + sid

    pltpu.sync_copy(sa_hbm.at[wid], idx_sa)
    pltpu.sync_copy(da_hbm.at[wid], idx_da)
    pltpu.sync_copy(sb_hbm.at[wid], idx_sb)
    pltpu.sync_copy(db_hbm.at[wid], idx_db)

    _zero_vmem_2d(rows0, CHUNK, d)
    _zero_vmem_1d(vals, CHUNK)
    for j in range(rpt_a // CHUNK):
      pltpu.sync_copy(rows0, acca_sh.at[pl.ds(sid * rpt_a + j * CHUNK, CHUNK)])
      pltpu.sync_copy(vals, dega_sh.at[pl.ds(sid * rpt_a + j * CHUNK, CHUNK)])
    for j in range(rpt_b // CHUNK):
      pltpu.sync_copy(rows0, accb_sh.at[pl.ds(sid * rpt_b + j * CHUNK, CHUNK)])
      pltpu.sync_copy(vals, degb_sh.at[pl.ds(sid * rpt_b + j * CHUNK, CHUNK)])

    work = ([(ta_hbm, idx_sa, idx_da, acca_sh, dega_sh, per_w_a, e_real_a, c)
             for c in range(nc_a)] +
            [(tb_hbm, idx_sb, idx_db, accb_sh, degb_sh, per_w_b, e_real_b, c)
             for c in range(nc_b)])
    bufs = [rows0, rows1]
    sems = [sem0, sem1]
    ssems = [ssem0, ssem1]
    descs = [None, None]
    sdescs = [None, None]
    t0, s0, _, _, _, _, _, _ = work[0]
    descs[0] = pltpu.async_copy(t0.at[s0.at[0]], rows0, sem0)
    plsc.subcore_barrier()

    iota = lax.iota(jnp.int32, 16)
    for i, (tbl, isx, idx, acc_sh, deg_sh, per_w, e_real, c) in enumerate(work):
      b = i % 2
      descs[b].wait()
      if i + 1 < len(work):
        if sdescs[1 - b] is not None:
          sdescs[1 - b].wait()
          sdescs[1 - b] = None
        tn, sn, _, _, _, _, _, cn = work[i + 1]
        descs[1 - b] = pltpu.async_copy(tn.at[sn.at[cn]], bufs[1 - b],
                                        sems[1 - b])
      base = wid * per_w + c * CHUNK
      for j in range(CHUNK // 16):
        m = (base + 16 * j + iota) < e_real
        vals[pl.ds(16 * j, 16)] = jnp.where(m, 1.0, 0.0)
      sdescs[b] = pltpu.async_copy(bufs[b], acc_sh.at[idx.at[c]], ssems[b],
                                   add=True)
      pltpu.sync_copy(vals, deg_sh.at[idx.at[c]], add=True)
    for sd in sdescs:
      if sd is not None:
        sd.wait()
    plsc.subcore_barrier()

    pltpu.sync_copy(acca_sh.at[pl.ds(sid * rpt_a, rpt_a)],
                    acca_hbm.at[cid, pl.ds(sid * rpt_a, rpt_a)])
    pltpu.sync_copy(dega_sh.at[pl.ds(sid * rpt_a, rpt_a)],
                    dega_hbm.at[cid, pl.ds(sid * rpt_a, rpt_a)])
    pltpu.sync_copy(accb_sh.at[pl.ds(sid * rpt_b, rpt_b)],
                    accb_hbm.at[cid, pl.ds(sid * rpt_b, rpt_b)])
    pltpu.sync_copy(degb_sh.at[pl.ds(sid * rpt_b, rpt_b)],
                    degb_hbm.at[cid, pl.ds(sid * rpt_b, rpt_b)])

  fn = pl.kernel(
      body,
      out_type=(jax.ShapeDtypeStruct((NC, pad_a, d), jnp.float32),
                jax.ShapeDtypeStruct((NC, pad_a), jnp.float32),
                jax.ShapeDtypeStruct((NC, pad_b, d), jnp.float32),
                jax.ShapeDtypeStruct((NC, pad_b), jnp.float32)),
      mesh=_MESH,
      compiler_params=pltpu.CompilerParams(needs_layout_passes=False,
                                           use_tc_tiling_on_sc=(d % 128 == 0)),
      scratch_types=[
          pltpu.VMEM((nc_a, CHUNK), jnp.int32),
          pltpu.VMEM((nc_a, CHUNK), jnp.int32),
          pltpu.VMEM((nc_b, CHUNK), jnp.int32),
          pltpu.VMEM((nc_b, CHUNK), jnp.int32),
          pltpu.VMEM((CHUNK, d), jnp.float32),
          pltpu.VMEM((CHUNK, d), jnp.float32),
          pltpu.VMEM((CHUNK,), jnp.float32),
          pltpu.VMEM_SHARED((pad_a, d), jnp.float32),
          pltpu.VMEM_SHARED((pad_a,), jnp.float32),
          pltpu.VMEM_SHARED((pad_b, d), jnp.float32),
          pltpu.VMEM_SHARED((pad_b,), jnp.float32),
          pltpu.SemaphoreType.DMA,
          pltpu.SemaphoreType.DMA,
          pltpu.SemaphoreType.DMA,
          pltpu.SemaphoreType.DMA,
      ],
  )
  return fn(table_a, srcp_a.reshape(NW, nc_a, CHUNK),
            dstp_a.reshape(NW, nc_a, CHUNK),
            table_b, srcp_b.reshape(NW, nc_b, CHUNK),
            dstp_b.reshape(NW, nc_b, CHUNK))


@functools.partial(jax.jit, static_argnames=("n_dst", "e_pad", "e_real"))
def _sc_gat(hs_pad, ls_pad, ld, mvec, srcp, dstp, *, n_dst, e_pad, e_real):
  d = 128
  n_src_pad = hs_pad.shape[0]
  per_w = e_pad // NW
  n_chunks = per_w // CHUNK
  n_dst_pad = _pad_up(n_dst, NS * CHUNK)
  rows_per_tile = n_dst_pad // NS
  n_copies = rows_per_tile // CHUNK
  den_pad = n_dst_pad
  den_per_tile = rows_per_tile

  def body(hs_hbm, ls_hbm, ld_hbm, m_hbm, srcp_hbm, dstp_hbm,
           num_hbm, den_hbm,
           idx_s, idx_d, rows0, rows1, pbuf, ls_v, ld_v, m_v,
           num_sh, den_sh, sem0, sem1, ssem0, ssem1):
    cid = lax.axis_index("c")
    sid = lax.axis_index("s")
    wid = cid * NS + sid

    pltpu.sync_copy(srcp_hbm.at[wid], idx_s)
    pltpu.sync_copy(dstp_hbm.at[wid], idx_d)
    pltpu.sync_copy(ls_hbm, ls_v)
    pltpu.sync_copy(ld_hbm, ld_v)
    pltpu.sync_copy(m_hbm, m_v)

    _zero_vmem_2d(rows0, CHUNK, d)
    _zero_vmem_1d(pbuf, CHUNK)
    for j in range(n_copies):
      pltpu.sync_copy(rows0,
                      num_sh.at[pl.ds(sid * rows_per_tile + j * CHUNK, CHUNK)])
    for j in range(den_per_tile // CHUNK):
      pltpu.sync_copy(pbuf, den_sh.at[pl.ds(sid * den_per_tile + j * CHUNK, CHUNK)])

    bufs = [rows0, rows1]
    sems = [sem0, sem1]
    ssems = [ssem0, ssem1]
    descs = [None, None]
    sdescs = [None, None]
    descs[0] = pltpu.async_copy(hs_hbm.at[idx_s.at[0]], rows0, sem0)
    plsc.subcore_barrier()

    iota = lax.iota(jnp.int32, 16)
    m16 = m_v[...]
    for c in range(n_chunks):
      b = c % 2
      descs[b].wait()
      if c + 1 < n_chunks:
        if sdescs[1 - b] is not None:
          sdescs[1 - b].wait()
          sdescs[1 - b] = None
        descs[1 - b] = pltpu.async_copy(hs_hbm.at[idx_s.at[c + 1]],
                                        bufs[1 - b], sems[1 - b])
      base = wid * per_w + c * CHUNK
      for j in range(CHUNK // 16):
        sv = idx_s[c, pl.ds(16 * j, 16)]
        dv = idx_d[c, pl.ds(16 * j, 16)]
        lsg = plsc.load_gather(ls_v, [sv])
        ldg = plsc.load_gather(ld_v, [dv])
        t = lsg + ldg
        a = jnp.maximum(t, 0.2 * t)
        pv = jnp.exp(a - m16)
        m = (base + 16 * j + iota) < e_real
        pbuf[pl.ds(16 * j, 16)] = jnp.where(m, pv, 0.0)

      rows_c = bufs[b]

      def scale_row(g, _):
        for u in range(2):
          r = 2 * g + u
          pb = plsc.load_gather(pbuf, [jnp.full((16,), r, jnp.int32)])
          for k in range(d // 16):
            rows_c[r, pl.ds(16 * k, 16)] = rows_c[r, pl.ds(16 * k, 16)] * pb
        return 0

      lax.fori_loop(0, CHUNK // 2, scale_row, 0)
      sdescs[b] = pltpu.async_copy(rows_c, num_sh.at[idx_d.at[c]], ssems[b],
                                   add=True)
      pltpu.sync_copy(pbuf, den_sh.at[idx_d.at[c]], add=True)
    for sd in sdescs:
      if sd is not None:
        sd.wait()
    plsc.subcore_barrier()

    pltpu.sync_copy(
        num_sh.at[pl.ds(sid * rows_per_tile, rows_per_tile)],
        num_hbm.at[cid, pl.ds(sid * rows_per_tile, rows_per_tile)])
    pltpu.sync_copy(
        den_sh.at[pl.ds(sid * den_per_tile, den_per_tile)],
        den_hbm.at[cid, pl.ds(sid * den_per_tile, den_per_tile)])

  fn = pl.kernel(
      body,
      out_type=(jax.ShapeDtypeStruct((NC, n_dst_pad, d), jnp.float32),
                jax.ShapeDtypeStruct((NC, den_pad), jnp.float32)),
      mesh=_MESH,
      compiler_params=pltpu.CompilerParams(needs_layout_passes=False),
      scratch_types=[
          pltpu.VMEM((n_chunks, CHUNK), jnp.int32),
          pltpu.VMEM((n_chunks, CHUNK), jnp.int32),
          pltpu.VMEM((CHUNK, d), jnp.float32),
          pltpu.VMEM((CHUNK, d), jnp.float32),
          pltpu.VMEM((CHUNK,), jnp.float32),
          pltpu.VMEM((ls_pad.shape[0],), jnp.float32),
          pltpu.VMEM((ld.shape[0],), jnp.float32),
          pltpu.VMEM((16,), jnp.float32),
          pltpu.VMEM_SHARED((n_dst_pad, d), jnp.float32),
          pltpu.VMEM_SHARED((den_pad,), jnp.float32),
          pltpu.SemaphoreType.DMA,
          pltpu.SemaphoreType.DMA,
          pltpu.SemaphoreType.DMA,
          pltpu.SemaphoreType.DMA,
      ],
  )
  return fn(hs_pad, ls_pad, ld, mvec, srcp.reshape(NW, n_chunks, CHUNK),
            dstp.reshape(NW, n_chunks, CHUNK))


def _tc_a_body(xrv_ref, xr_ref, ws_ref, wd_ref, as_ref, ad_ref,
               hs_ref, ls_ref, ld_ref, m_ref):
  hs = jnp.dot(xrv_ref[...], ws_ref[...], preferred_element_type=jnp.float32)
  hs_ref[...] = hs
  ls = jnp.sum(hs * as_ref[...], axis=1, keepdims=True)
  ls_ref[...] = ls
  wv = jnp.dot(wd_ref[...], ad_ref[...].T, preferred_element_type=jnp.float32)
  ld = jnp.dot(xr_ref[...], wv, preferred_element_type=jnp.float32)
  ld_ref[...] = ld
  t = jnp.max(ls) + jnp.max(ld)
  m_ref[...] = jnp.full((1, 128), jnp.maximum(t, 0.2 * t), jnp.float32)


@jax.jit
def _tc_a(xrv, xr, ws, wd, att_s, att_d):
  n_rv, dd = xrv.shape
  n_r = xr.shape[0]
  return pl.pallas_call(
      _tc_a_body,
      out_shape=(jax.ShapeDtypeStruct((n_rv, dd), jnp.float32),
                 jax.ShapeDtypeStruct((n_rv, 1), jnp.float32),
                 jax.ShapeDtypeStruct((n_r, 1), jnp.float32),
                 jax.ShapeDtypeStruct((1, 128), jnp.float32)),
  )(xrv, xr, ws, wd, att_s.reshape(1, -1), att_d.reshape(1, -1))


def _tc_b1a_body(na_ref, nb_ref, da_ref, db_ref, bg_ref, rg_ref):
  den = da_ref[...] + db_ref[...]
  rg_ref[...] = (na_ref[...] + nb_ref[...]) / (den + 1e-16) + bg_ref[...]


@jax.jit
def _tc_b1a(num_a, num_b, den_a, den_b, bg):
  n, d = num_a.shape
  blk = 2000
  row = lambda i: (i, 0)
  full = lambda i: (0, 0)
  rspec = pl.BlockSpec((blk, d), row)
  cspec = pl.BlockSpec((blk, 1), row)
  return pl.pallas_call(
      _tc_b1a_body,
      grid=(n // blk,),
      in_specs=[rspec, rspec, cspec, cspec, pl.BlockSpec((1, d), full)],
      out_specs=rspec,
      out_shape=jax.ShapeDtypeStruct((n, d), jnp.float32),
  )(num_a, num_b, den_a, den_b, bg)


def _tc_b1b_body(rg_ref, a1a_ref, a1b_ref, g1a_ref, g1b_ref, wl_ref, wr_ref,
                 b0_ref, r0_ref):
  deg = jnp.maximum(g1a_ref[...] + g1b_ref[...], 1.0)
  m1 = (a1a_ref[...] + a1b_ref[...]) / deg
  r0 = (jnp.dot(m1, wl_ref[...], preferred_element_type=jnp.float32)
        + jnp.dot(rg_ref[...], wr_ref[...], preferred_element_type=jnp.float32)
        + b0_ref[...])
  r0_ref[...] = jnp.maximum(r0, 0.0)


@jax.jit
def _tc_b1b(rg, a1a, a1b, g1a, g1b, wl, wr, b0):
  n, d = rg.shape
  blk = 2000
  row = lambda i: (i, 0)
  full = lambda i: (0, 0)
  rspec = pl.BlockSpec((blk, d), row)
  cspec = pl.BlockSpec((blk, 1), row)
  wspec = pl.BlockSpec((d, d), full)
  return pl.pallas_call(
      _tc_b1b_body,
      grid=(n // blk,),
      in_specs=[rspec, rspec, rspec, cspec, cspec, wspec, wspec,
                pl.BlockSpec((1, d), full)],
      out_specs=rspec,
      out_shape=jax.ShapeDtypeStruct((n, d), jnp.float32),
  )(rg, a1a, a1b, g1a, g1b, wl, wr, b0)


def _tc_b2_body(a2a_ref, a2b_ref, g2a_ref, g2b_ref, a3a_ref, a3b_ref,
                g3a_ref, g3b_ref, xrr_ref, r0_ref,
                wl_a_ref, wr_a_ref, ba_ref, wl_b_ref, wr_b_ref, bb_ref,
                wl1_ref, wr1_ref, z_ref, y_ref):
  deg2 = jnp.maximum(g2a_ref[...] + g2b_ref[...], 1.0)
  m2 = (a2a_ref[...] + a2b_ref[...]) / deg2
  deg3 = jnp.maximum(g3a_ref[...] + g3b_ref[...], 1.0)
  m3 = (a3a_ref[...] + a3b_ref[...]) / deg3
  xrr = xrr_ref[...]
  rr = (jnp.dot(m2, wl_a_ref[...], preferred_element_type=jnp.float32)
        + jnp.dot(xrr, wr_a_ref[...], preferred_element_type=jnp.float32)
        + ba_ref[...]
        + jnp.dot(m3, wl_b_ref[...], preferred_element_type=jnp.float32)
        + jnp.dot(xrr, wr_b_ref[...], preferred_element_type=jnp.float32)
        + bb_ref[...])
  rr0 = jnp.maximum(0.5 * rr, 0.0)
  z_ref[...] = jnp.dot(rr0, wl1_ref[...], preferred_element_type=jnp.float32)
  y_ref[...] = jnp.dot(r0_ref[...], wr1_ref[...], preferred_element_type=jnp.float32)


@jax.jit
def _tc_b2(a2a, a2b, g2a, g2b, a3a, a3b, g3a, g3b, xrr, r0,
           wl_a, wr_a, ba, wl_b, wr_b, bb, wl1, wr1):
  n_rr, d = xrr.shape
  n_r = r0.shape[0]
  do = wl1.shape[1]
  return pl.pallas_call(
      _tc_b2_body,
      out_shape=(jax.ShapeDtypeStruct((n_rr, do), jnp.float32),
                 jax.ShapeDtypeStruct((n_r, do), jnp.float32)),
  )(a2a, a2b, g2a, g2b, a3a, a3b, g3a, g3b, xrr, r0,
    wl_a, wr_a, ba, wl_b, wr_b, bb, wl1, wr1)


def _tc_b3_body(a4a_ref, a4b_ref, g1a_ref, g1b_ref, y_ref, b1_ref, r1_ref):
  deg = jnp.maximum(g1a_ref[...] + g1b_ref[...], 1.0)
  r1_ref[...] = (a4a_ref[...] + a4b_ref[...]) / deg + y_ref[...] + b1_ref[...]


@jax.jit
def _tc_b3(a4a, a4b, g1a, g1b, y, b1):
  n, do = y.shape
  blk = 2000
  grid = n // blk
  row = lambda i: (i, 0)
  full = lambda i: (0, 0)
  rspec = pl.BlockSpec((blk, do), row)
  cspec = pl.BlockSpec((blk, 1), row)
  return pl.pallas_call(
      _tc_b3_body,
      grid=(grid,),
      in_specs=[rspec, rspec, cspec, cspec, rspec,
                pl.BlockSpec((1, do), full)],
      out_specs=rspec,
      out_shape=jax.ShapeDtypeStruct((n, do), jnp.float32),
  )(a4a, a4b, g1a, g1b, y, b1)


def _tc_b4_body(a5a_ref, a5b_ref, g5a_ref, g5b_ref, o_ref):
  deg = jnp.maximum(g5a_ref[...] + g5b_ref[...], 1.0)
  o_ref[...] = (a5a_ref[...] + a5b_ref[...]) / deg


@jax.jit
def _tc_b4(a5a, a5b, g5a, g5b):
  n, do = a5a.shape
  return pl.pallas_call(
      _tc_b4_body,
      out_shape=jax.ShapeDtypeStruct((n, do), jnp.float32),
  )(a5a, a5b, g5a, g5b)


def _zpad(x, rows=PAD_ROWS):
  return jnp.concatenate([x, jnp.zeros((rows,) + x.shape[1:], x.dtype)], axis=0)


def kernel(x_object, x_room, x_object_virtual, x_room_virtual, x_or, x_rr,
           params, e_ov_o, e_rv_r, e_o_or, e_or_o, e_r_rr, e_rr_r, e_or_rr,
           e_rr_or, e_r_rv):
  pre = params['pre_rv_r']
  s0 = params['sage0']
  s1 = params['sage1']
  n_room = x_room.shape[0]
  n_rr = x_rr.shape[0]
  n_rv = x_room_virtual.shape[0]
  n_or = x_or.shape[0]

  hs, ls, ld, m = _tc_a(x_room_virtual, x_room, pre['Ws'], pre['Wd'],
                        pre['att_s'], pre['att_d'])
  hs_pad = _zpad(hs)
  ls_pad = jnp.concatenate([ls[:, 0], jnp.zeros((PAD_ROWS,), jnp.float32)])
  mvec = m[0, :16]

  src_gat, dst_gat, ep_gat = _pad_edges(e_rv_r, n_rv)
  src_rrr, dst_rrr, ep_rrr = _pad_edges(e_rr_r, n_rr)
  src_r2r, dst_r2r, ep_r2r = _pad_edges(e_r_rr, n_room)
  src_orr, dst_orr, ep_orr = _pad_edges(e_or_rr, n_or)
  src_rv, dst_rv, ep_rv = _pad_edges(e_r_rv, n_room)

  agg1, deg1 = _sc_seg_sum(_zpad(x_rr), src_rrr, dst_rrr,
                           n_dst=n_room, d=128, e_pad=ep_rrr,
                           e_real=e_rr_r.shape[1])
  num, den = _sc_gat(hs_pad, ls_pad, ld[:, 0], mvec, src_gat, dst_gat,
                     n_dst=n_room, e_pad=ep_gat, e_real=e_rv_r.shape[1])

  rg = _tc_b1a(num[0, :n_room], num[1, :n_room],
               den[0, :n_room, None], den[1, :n_room, None],
               pre['b'].reshape(1, -1))

  agg2, deg2, agg3, deg3 = _sc_seg_sum2(
      _zpad(rg), src_r2r, dst_r2r, _zpad(x_or), src_orr, dst_orr,
      n_dst_a=n_rr, n_dst_b=n_rr, d=128,
      e_pad_a=ep_r2r, e_real_a=e_r_rr.shape[1],
      e_pad_b=ep_orr, e_real_b=e_or_rr.shape[1])

  r0 = _tc_b1b(rg, agg1[0, :n_room], agg1[1, :n_room],
               deg1[0, :n_room, None], deg1[1, :n_room, None],
               s0['rr_r']['Wl'], s0['rr_r']['Wr'],
               s0['rr_r']['b'].reshape(1, -1))

  z, y = _tc_b2(agg2[0, :n_rr], agg2[1, :n_rr],
                deg2[0, :n_rr, None], deg2[1, :n_rr, None],
                agg3[0, :n_rr], agg3[1, :n_rr],
                deg3[0, :n_rr, None], deg3[1, :n_rr, None],
                x_rr, r0,
                s0['r_rr']['Wl'], s0['r_rr']['Wr'], s0['r_rr']['b'].reshape(1, -1),
                s0['or_rr']['Wl'], s0['or_rr']['Wr'], s0['or_rr']['b'].reshape(1, -1),
                s1['rr_r']['Wl'], s1['rr_r']['Wr'])

  agg4, _ = _sc_seg_sum(_zpad(z), src_rrr, dst_rrr,
                        n_dst=n_room, d=32, e_pad=ep_rrr,
                        e_real=e_rr_r.shape[1])

  r1 = _tc_b3(agg4[0, :n_room], agg4[1, :n_room],
              deg1[0, :n_room, None], deg1[1, :n_room, None],
              y, s1['rr_r']['b'].reshape(1, -1))

  agg5, deg5 = _sc_seg_sum(_zpad(r1), src_rv, dst_rv,
                           n_dst=n_rv, d=32, e_pad=ep_rv,
                           e_real=e_r_rv.shape[1])

  return _tc_b4(agg5[0, :n_rv], agg5[1, :n_rv],
                deg5[0, :n_rv, None], deg5[1, :n_rv, None])
